# baseline (speedup 1.0000x reference)
import functools

import jax
import jax.numpy as jnp
from jax import lax
from jax.experimental import pallas as pl
from jax.experimental.pallas import tpu as pltpu
from jax.experimental.pallas import tpu_sc as plsc

_NC = 2
_NS = 16
_NW = _NC * _NS
_C = 128
_NB = 4


def _pad_edges(edge_index, n):
    e = edge_index.shape[1]
    quantum = _NW * _C
    e_pad = -(-e // quantum) * quantum
    if e_pad == e:
        return edge_index, e
    extra = e_pad - e
    dummy = jnp.stack([
        jnp.zeros((extra,), edge_index.dtype),
        jnp.full((extra,), n, edge_index.dtype),
    ])
    return jnp.concatenate([edge_index, dummy], axis=1), e_pad


def _acc_slabs(nacc):
    rpt = (nacc // _NS) // 8 * 8
    rem = nacc - _NS * rpt
    assert rem % 8 == 0
    return rpt, rem


def _zero_acc(zeros_hbm, acc, s, rpt, rem):
    pltpu.sync_copy(zeros_hbm.at[pl.ds(0, rpt)],
                    acc.at[pl.ds(s * rpt, rpt)])
    if rem:
        @pl.when(s == _NS - 1)
        def _():
            pltpu.sync_copy(zeros_hbm.at[pl.ds(0, rem)],
                            acc.at[pl.ds(_NS * rpt, rem)])


def _write_acc(acc, out_hbm, c, s, n):
    rpt, rem = _acc_slabs(n)
    pltpu.sync_copy(acc.at[pl.ds(s * rpt, rpt)],
                    out_hbm.at[c].at[pl.ds(s * rpt, rpt)])
    if rem:
        @pl.when(s == _NS - 1)
        def _():
            pltpu.sync_copy(acc.at[pl.ds(_NS * rpt, rem)],
                            out_hbm.at[c].at[pl.ds(_NS * rpt, rem)])


def _sc_scatter_sum(table, edge_index, zeros, n):
    d = table.shape[1]
    nacc = n + 8
    e = edge_index.shape[1]
    assert e % _NW == 0
    epw = e // _NW
    assert epw % _C == 0
    nchunk = epw // _C
    tail = nchunk % _NB
    groups = nchunk // _NB
    assert nchunk >= _NB
    rpt, rem = _acc_slabs(nacc)

    mesh = plsc.VectorSubcoreMesh(core_axis_name="c", subcore_axis_name="s")

    @functools.partial(
        pl.kernel,
        mesh=mesh,
        name="sc_scatter_sum",
        out_type=jax.ShapeDtypeStruct((_NC, n, d), jnp.float32),
        scratch_types=[
            pltpu.VMEM((epw,), jnp.int32),
            pltpu.VMEM((epw,), jnp.int32),
            [pltpu.VMEM((_C, d), jnp.float32)] * _NB,
            [pltpu.SemaphoreType.DMA] * _NB,
            [pltpu.SemaphoreType.DMA] * _NB,
            pltpu.VMEM_SHARED((nacc, d), jnp.float32),
        ],
        compiler_params=pltpu.CompilerParams(use_tc_tiling_on_sc=False),
    )
    def k(table_hbm, ei_hbm, zeros_hbm, out_hbm,
          srcs, dsts, rows, gsem, ssem, acc):
        c = lax.axis_index("c")
        s = lax.axis_index("s")
        w = c * _NS + s

        pltpu.sync_copy(ei_hbm.at[0].at[pl.ds(w * epw, epw)], srcs)
        pltpu.sync_copy(ei_hbm.at[1].at[pl.ds(w * epw, epw)], dsts)
        _zero_acc(zeros_hbm, acc, s, rpt, rem)
        plsc.subcore_barrier()

        def fire_gather(i, b):
            pltpu.async_copy(
                table_hbm.at[srcs.at[pl.ds(i * _C, _C)]], rows[b], gsem[b])

        def wait_gather(i, b):
            pltpu.make_async_copy(
                table_hbm.at[srcs.at[pl.ds(i * _C, _C)]], rows[b], gsem[b]
            ).wait()

        def fire_scatter(i, b):
            pltpu.async_copy(
                rows[b], acc.at[dsts.at[pl.ds(i * _C, _C)]], ssem[b],
                add=True)

        def wait_scatter(i, b):
            pltpu.make_async_copy(
                rows[b], acc.at[dsts.at[pl.ds(i * _C, _C)]], ssem[b]
            ).wait()

        for b in range(_NB):
            fire_gather(b, b)

        @pl.loop(0, groups)
        def _(g):
            i0 = g * _NB
            for b in range(_NB):
                i = i0 + b
                wait_gather(i, b)
                fire_scatter(i, b)
            for b in range(_NB):
                i = i0 + b

                @pl.when(i + _NB < nchunk)
                def _():
                    wait_scatter(i, b)
                    fire_gather(i + _NB, b)

        for b in range(tail):
            i = groups * _NB + b
            wait_gather(i, b)
            fire_scatter(i, b)
        for b in range(_NB):
            i = groups * _NB + b if b < tail else (groups - 1) * _NB + b
            wait_scatter(i, b)

        plsc.subcore_barrier()
        _write_acc(acc, out_hbm, c, s, n)

    return k(table, edge_index, zeros)


def _sc_degree(edge_index, zeros, n):
    d = zeros.shape[1]
    nacc = n + 8
    e = edge_index.shape[1]
    assert e % _NW == 0
    epw = e // _NW
    assert epw % _C == 0
    nchunk = epw // _C
    rpt, rem = _acc_slabs(nacc)

    mesh = plsc.VectorSubcoreMesh(core_axis_name="c", subcore_axis_name="s")

    @functools.partial(
        pl.kernel,
        mesh=mesh,
        name="sc_degree",
        out_type=jax.ShapeDtypeStruct((_NC, n, d), jnp.float32),
        scratch_types=[
            pltpu.VMEM((epw,), jnp.int32),
            pltpu.VMEM((_C, d), jnp.float32),
            pltpu.VMEM_SHARED((nacc, d), jnp.float32),
            pltpu.SemaphoreType.DMA,
        ],
        compiler_params=pltpu.CompilerParams(use_tc_tiling_on_sc=False),
    )
    def k(ei_hbm, zeros_hbm, out_hbm, dsts, ones, acc, ssem):
        c = lax.axis_index("c")
        s = lax.axis_index("s")
        w = c * _NS + s

        pltpu.sync_copy(ei_hbm.at[1].at[pl.ds(w * epw, epw)], dsts)

        @pl.loop(0, _C)
        def _(j):
            ones[j] = jnp.full((d,), 1.0, jnp.float32)

        _zero_acc(zeros_hbm, acc, s, rpt, rem)
        plsc.subcore_barrier()

        @pl.loop(0, nchunk)
        def _(i):
            pltpu.async_copy(ones, acc.at[dsts.at[pl.ds(i * _C, _C)]],
                             ssem, add=True)

        @pl.loop(0, nchunk)
        def _(i):
            pltpu.make_async_copy(
                ones, acc.at[dsts.at[pl.ds(i * _C, _C)]], ssem).wait()

        plsc.subcore_barrier()
        _write_acc(acc, out_hbm, c, s, n)

    return k(edge_index, zeros)


_BLK = 2000


def _tc_layer1(x, w1, degp):
    n, din = x.shape
    dh = w1.shape[1]
    dd = degp.shape[2]

    def body(x_ref, w_ref, dp_ref, y_ref, dinv_ref):
        deg = dp_ref[0, :, 0:1] + dp_ref[1, :, 0:1] + 1.0
        dinv = lax.rsqrt(deg)
        dinv_ref[...] = dinv
        xw = jnp.dot(x_ref[...], w_ref[...],
                     preferred_element_type=jnp.float32)
        y_ref[...] = xw * dinv

    grid = n // _BLK
    return pl.pallas_call(
        body,
        name="tc_layer1",
        grid=(grid,),
        in_specs=[
            pl.BlockSpec((_BLK, din), lambda i: (i, 0)),
            pl.BlockSpec((din, dh), lambda i: (0, 0)),
            pl.BlockSpec((_NC, _BLK, dd), lambda i: (0, i, 0)),
        ],
        out_specs=(
            pl.BlockSpec((_BLK, dh), lambda i: (i, 0)),
            pl.BlockSpec((_BLK, 1), lambda i: (i, 0)),
        ),
        out_shape=(
            jax.ShapeDtypeStruct((n, dh), jnp.float32),
            jax.ShapeDtypeStruct((n, 1), jnp.float32),
        ),
    )(x, w1, degp)


def _tc_mid(a, y1, dinv, b1):
    n, dh = y1.shape

    def body(a_ref, y_ref, dinv_ref, b_ref, u_ref):
        total = a_ref[0] + a_ref[1] + y_ref[...]
        h = jnp.maximum(dinv_ref[...] * total + b_ref[...], 0.0)
        u_ref[...] = dinv_ref[...] * h

    grid = n // _BLK
    return pl.pallas_call(
        body,
        name="tc_mid",
        grid=(grid,),
        in_specs=[
            pl.BlockSpec((_NC, _BLK, dh), lambda i: (0, i, 0)),
            pl.BlockSpec((_BLK, dh), lambda i: (i, 0)),
            pl.BlockSpec((_BLK, 1), lambda i: (i, 0)),
            pl.BlockSpec((1, dh), lambda i: (0, 0)),
        ],
        out_specs=pl.BlockSpec((_BLK, dh), lambda i: (i, 0)),
        out_shape=jax.ShapeDtypeStruct((n, dh), jnp.float32),
    )(a, y1, dinv, b1)


def _tc_out(a, u, dinv, w2, b2):
    n, dh = u.shape
    dout = w2.shape[1]

    def body(a_ref, u_ref, dinv_ref, w_ref, b_ref, o_ref):
        m = dinv_ref[...] * (a_ref[0] + a_ref[1] + u_ref[...])
        o_ref[...] = jnp.dot(m, w_ref[...],
                             preferred_element_type=jnp.float32) + b_ref[...]

    grid = n // _BLK
    return pl.pallas_call(
        body,
        name="tc_out",
        grid=(grid,),
        in_specs=[
            pl.BlockSpec((_NC, _BLK, dh), lambda i: (0, i, 0)),
            pl.BlockSpec((_BLK, dh), lambda i: (i, 0)),
            pl.BlockSpec((_BLK, 1), lambda i: (i, 0)),
            pl.BlockSpec((dh, dout), lambda i: (0, 0)),
            pl.BlockSpec((1, dout), lambda i: (0, 0)),
        ],
        out_specs=pl.BlockSpec((_BLK, dout), lambda i: (i, 0)),
        out_shape=jax.ShapeDtypeStruct((n, dout), jnp.float32),
    )(a, u, dinv, w2, b2)


def kernel(x, edge_index, W1, b1, W2, b2):
    n = x.shape[0]
    dh = W1.shape[1]

    ei, _ = _pad_edges(edge_index, n)

    rpt, rem = _acc_slabs(n + 8)
    zrows = max(rpt, rem)
    zeros_h = jnp.zeros((zrows, dh), jnp.float32)
    zeros_d = jnp.zeros((zrows, 16), jnp.float32)

    degp = _sc_degree(ei, zeros_d, n)

    y1, dinv = _tc_layer1(x, W1, degp)
    a1 = _sc_scatter_sum(y1, ei, zeros_h, n)
    u = _tc_mid(a1, y1, dinv, b1.reshape(1, dh))
    a2 = _sc_scatter_sum(u, ei, zeros_h, n)
    return _tc_out(a2, u, dinv, W2, b2.reshape(1, W2.shape[1]))


# speedup vs baseline: 1.7261x; 1.7261x over previous
import functools

import jax
import jax.numpy as jnp
from jax import lax
from jax.experimental import pallas as pl
from jax.experimental.pallas import tpu as pltpu
from jax.experimental.pallas import tpu_sc as plsc

_NC = 2
_NS = 16
_NW = _NC * _NS
_C = 80
_SPARE = 128
_NB = 4


def _pad_edges(edge_index, n):
    e = edge_index.shape[1]
    quantum = _NW * _C
    e_pad = -(-e // quantum) * quantum
    if e_pad == e:
        return edge_index, e
    extra = e_pad - e
    dummy = jnp.stack([
        jnp.zeros((extra,), edge_index.dtype),
        n + (jnp.arange(extra, dtype=edge_index.dtype) % _SPARE),
    ])
    return jnp.concatenate([edge_index, dummy], axis=1), e_pad


def _acc_slabs(nacc):
    rpt = (nacc // _NS) // 8 * 8
    rem = nacc - _NS * rpt
    assert rem % 8 == 0
    return rpt, rem


def _zero_acc(zeros_hbm, acc, s, rpt, rem):
    pltpu.sync_copy(zeros_hbm.at[pl.ds(0, rpt)],
                    acc.at[pl.ds(s * rpt, rpt)])
    if rem:
        @pl.when(s == _NS - 1)
        def _():
            pltpu.sync_copy(zeros_hbm.at[pl.ds(0, rem)],
                            acc.at[pl.ds(_NS * rpt, rem)])


def _write_acc(acc, out_hbm, c, s, n):
    rpt, rem = _acc_slabs(n)
    pltpu.sync_copy(acc.at[pl.ds(s * rpt, rpt)],
                    out_hbm.at[c].at[pl.ds(s * rpt, rpt)])
    if rem:
        @pl.when(s == _NS - 1)
        def _():
            pltpu.sync_copy(acc.at[pl.ds(_NS * rpt, rem)],
                            out_hbm.at[c].at[pl.ds(_NS * rpt, rem)])


def _sc_scatter_sum(table, edge_index, zeros, n):
    d = table.shape[1]
    nacc = n + _SPARE
    e = edge_index.shape[1]
    assert e % _NW == 0
    epw = e // _NW
    assert epw % _C == 0
    nchunk = epw // _C
    tail = nchunk % _NB
    groups = nchunk // _NB
    assert nchunk >= _NB
    rpt, rem = _acc_slabs(nacc)

    mesh = plsc.VectorSubcoreMesh(core_axis_name="c", subcore_axis_name="s")

    @functools.partial(
        pl.kernel,
        mesh=mesh,
        name="sc_scatter_sum",
        out_type=jax.ShapeDtypeStruct((_NC, n, d), jnp.float32),
        scratch_types=[
            pltpu.VMEM((epw,), jnp.int32),
            pltpu.VMEM((epw,), jnp.int32),
            [pltpu.VMEM((_C, d), jnp.float32)] * _NB,
            [pltpu.SemaphoreType.DMA] * _NB,
            [pltpu.SemaphoreType.DMA] * _NB,
            pltpu.VMEM_SHARED((nacc, d), jnp.float32),
        ],
        compiler_params=pltpu.CompilerParams(use_tc_tiling_on_sc=False),
    )
    def k(table_hbm, ei_hbm, zeros_hbm, out_hbm,
          srcs, dsts, rows, gsem, ssem, acc):
        c = lax.axis_index("c")
        s = lax.axis_index("s")
        w = c * _NS + s

        pltpu.sync_copy(ei_hbm.at[0].at[pl.ds(w * epw, epw)], srcs)
        pltpu.sync_copy(ei_hbm.at[1].at[pl.ds(w * epw, epw)], dsts)
        _zero_acc(zeros_hbm, acc, s, rpt, rem)
        plsc.subcore_barrier()

        def fire_gather(i, b):
            pltpu.async_copy(
                table_hbm.at[srcs.at[pl.ds(i * _C, _C)]], rows[b], gsem[b])

        def wait_gather(i, b):
            pltpu.make_async_copy(
                table_hbm.at[srcs.at[pl.ds(i * _C, _C)]], rows[b], gsem[b]
            ).wait()

        def fire_scatter(i, b):
            pltpu.async_copy(
                rows[b], acc.at[dsts.at[pl.ds(i * _C, _C)]], ssem[b],
                add=True)

        def wait_scatter(i, b):
            pltpu.make_async_copy(
                rows[b], acc.at[dsts.at[pl.ds(i * _C, _C)]], ssem[b]
            ).wait()

        for b in range(_NB):
            fire_gather(b, b)

        @pl.loop(0, groups)
        def _(g):
            i0 = g * _NB
            for b in range(_NB):
                i = i0 + b
                wait_gather(i, b)
                fire_scatter(i, b)
            for b in range(_NB):
                i = i0 + b

                @pl.when(i + _NB < nchunk)
                def _():
                    wait_scatter(i, b)
                    fire_gather(i + _NB, b)

        for b in range(tail):
            i = groups * _NB + b
            wait_gather(i, b)
            fire_scatter(i, b)
        for b in range(_NB):
            i = groups * _NB + b if b < tail else (groups - 1) * _NB + b
            wait_scatter(i, b)

        plsc.subcore_barrier()
        _write_acc(acc, out_hbm, c, s, n)

    return k(table, edge_index, zeros)


def _sc_degree(edge_index, zeros, n):
    d = zeros.shape[1]
    nacc = n + _SPARE
    e = edge_index.shape[1]
    assert e % _NW == 0
    epw = e // _NW
    assert epw % _C == 0
    nchunk = epw // _C
    rpt, rem = _acc_slabs(nacc)

    mesh = plsc.VectorSubcoreMesh(core_axis_name="c", subcore_axis_name="s")

    @functools.partial(
        pl.kernel,
        mesh=mesh,
        name="sc_degree",
        out_type=jax.ShapeDtypeStruct((_NC, n, d), jnp.float32),
        scratch_types=[
            pltpu.VMEM((epw,), jnp.int32),
            pltpu.VMEM((_C, d), jnp.float32),
            pltpu.VMEM_SHARED((nacc, d), jnp.float32),
            pltpu.SemaphoreType.DMA,
        ],
        compiler_params=pltpu.CompilerParams(use_tc_tiling_on_sc=False),
    )
    def k(ei_hbm, zeros_hbm, out_hbm, dsts, ones, acc, ssem):
        c = lax.axis_index("c")
        s = lax.axis_index("s")
        w = c * _NS + s

        pltpu.sync_copy(ei_hbm.at[1].at[pl.ds(w * epw, epw)], dsts)

        @pl.loop(0, _C)
        def _(j):
            ones[j] = jnp.full((d,), 1.0, jnp.float32)

        _zero_acc(zeros_hbm, acc, s, rpt, rem)
        plsc.subcore_barrier()

        @pl.loop(0, nchunk)
        def _(i):
            pltpu.async_copy(ones, acc.at[dsts.at[pl.ds(i * _C, _C)]],
                             ssem, add=True)

        @pl.loop(0, nchunk)
        def _(i):
            pltpu.make_async_copy(
                ones, acc.at[dsts.at[pl.ds(i * _C, _C)]], ssem).wait()

        plsc.subcore_barrier()
        _write_acc(acc, out_hbm, c, s, n)

    return k(edge_index, zeros)


_BLK = 2000


def _tc_layer1(x, w1, degp):
    n, din = x.shape
    dh = w1.shape[1]
    dd = degp.shape[2]

    def body(x_ref, w_ref, dp_ref, y_ref, dinv_ref):
        deg = dp_ref[0, :, 0:1] + dp_ref[1, :, 0:1] + 1.0
        dinv = lax.rsqrt(deg)
        dinv_ref[...] = dinv
        xw = jnp.dot(x_ref[...], w_ref[...],
                     preferred_element_type=jnp.float32)
        y_ref[...] = xw * dinv

    grid = n // _BLK
    return pl.pallas_call(
        body,
        name="tc_layer1",
        grid=(grid,),
        in_specs=[
            pl.BlockSpec((_BLK, din), lambda i: (i, 0)),
            pl.BlockSpec((din, dh), lambda i: (0, 0)),
            pl.BlockSpec((_NC, _BLK, dd), lambda i: (0, i, 0)),
        ],
        out_specs=(
            pl.BlockSpec((_BLK, dh), lambda i: (i, 0)),
            pl.BlockSpec((_BLK, 1), lambda i: (i, 0)),
        ),
        out_shape=(
            jax.ShapeDtypeStruct((n, dh), jnp.float32),
            jax.ShapeDtypeStruct((n, 1), jnp.float32),
        ),
    )(x, w1, degp)


def _tc_mid(a, y1, dinv, b1):
    n, dh = y1.shape

    def body(a_ref, y_ref, dinv_ref, b_ref, u_ref):
        total = a_ref[0] + a_ref[1] + y_ref[...]
        h = jnp.maximum(dinv_ref[...] * total + b_ref[...], 0.0)
        u_ref[...] = dinv_ref[...] * h

    grid = n // _BLK
    return pl.pallas_call(
        body,
        name="tc_mid",
        grid=(grid,),
        in_specs=[
            pl.BlockSpec((_NC, _BLK, dh), lambda i: (0, i, 0)),
            pl.BlockSpec((_BLK, dh), lambda i: (i, 0)),
            pl.BlockSpec((_BLK, 1), lambda i: (i, 0)),
            pl.BlockSpec((1, dh), lambda i: (0, 0)),
        ],
        out_specs=pl.BlockSpec((_BLK, dh), lambda i: (i, 0)),
        out_shape=jax.ShapeDtypeStruct((n, dh), jnp.float32),
    )(a, y1, dinv, b1)


def _tc_out(a, u, dinv, w2, b2):
    n, dh = u.shape
    dout = w2.shape[1]

    def body(a_ref, u_ref, dinv_ref, w_ref, b_ref, o_ref):
        m = dinv_ref[...] * (a_ref[0] + a_ref[1] + u_ref[...])
        o_ref[...] = jnp.dot(m, w_ref[...],
                             preferred_element_type=jnp.float32) + b_ref[...]

    grid = n // _BLK
    return pl.pallas_call(
        body,
        name="tc_out",
        grid=(grid,),
        in_specs=[
            pl.BlockSpec((_NC, _BLK, dh), lambda i: (0, i, 0)),
            pl.BlockSpec((_BLK, dh), lambda i: (i, 0)),
            pl.BlockSpec((_BLK, 1), lambda i: (i, 0)),
            pl.BlockSpec((dh, dout), lambda i: (0, 0)),
            pl.BlockSpec((1, dout), lambda i: (0, 0)),
        ],
        out_specs=pl.BlockSpec((_BLK, dout), lambda i: (i, 0)),
        out_shape=jax.ShapeDtypeStruct((n, dout), jnp.float32),
    )(a, u, dinv, w2, b2)


def kernel(x, edge_index, W1, b1, W2, b2):
    n = x.shape[0]
    dh = W1.shape[1]

    ei, _ = _pad_edges(edge_index, n)

    rpt, rem = _acc_slabs(n + _SPARE)
    zrows = max(rpt, rem)
    zeros_h = jnp.zeros((zrows, dh), jnp.float32)
    zeros_d = jnp.zeros((zrows, 16), jnp.float32)

    degp = _sc_degree(ei, zeros_d, n)

    y1, dinv = _tc_layer1(x, W1, degp)
    a1 = _sc_scatter_sum(y1, ei, zeros_h, n)
    u = _tc_mid(a1, y1, dinv, b1.reshape(1, dh))
    a2 = _sc_scatter_sum(u, ei, zeros_h, n)
    return _tc_out(a2, u, dinv, W2, b2.reshape(1, W2.shape[1]))


# speedup vs baseline: 1.7686x; 1.0246x over previous
import functools

import jax
import jax.numpy as jnp
from jax import lax
from jax.experimental import pallas as pl
from jax.experimental.pallas import tpu as pltpu
from jax.experimental.pallas import tpu_sc as plsc

_NC = 2
_NS = 16
_NW = _NC * _NS
_C = 80
_SPARE = 128
_NB = 8


def _pad_edges(edge_index, n):
    e = edge_index.shape[1]
    quantum = _NW * _C
    e_pad = -(-e // quantum) * quantum
    if e_pad == e:
        return edge_index, e
    extra = e_pad - e
    dummy = jnp.stack([
        jnp.zeros((extra,), edge_index.dtype),
        n + (jnp.arange(extra, dtype=edge_index.dtype) % _SPARE),
    ])
    return jnp.concatenate([edge_index, dummy], axis=1), e_pad


def _acc_slabs(nacc):
    rpt = (nacc // _NS) // 8 * 8
    rem = nacc - _NS * rpt
    assert rem % 8 == 0
    return rpt, rem


def _zero_acc(zeros_hbm, acc, s, rpt, rem):
    pltpu.sync_copy(zeros_hbm.at[pl.ds(0, rpt)],
                    acc.at[pl.ds(s * rpt, rpt)])
    if rem:
        @pl.when(s == _NS - 1)
        def _():
            pltpu.sync_copy(zeros_hbm.at[pl.ds(0, rem)],
                            acc.at[pl.ds(_NS * rpt, rem)])


def _write_acc(acc, out_hbm, c, s, n):
    rpt, rem = _acc_slabs(n)
    pltpu.sync_copy(acc.at[pl.ds(s * rpt, rpt)],
                    out_hbm.at[c].at[pl.ds(s * rpt, rpt)])
    if rem:
        @pl.when(s == _NS - 1)
        def _():
            pltpu.sync_copy(acc.at[pl.ds(_NS * rpt, rem)],
                            out_hbm.at[c].at[pl.ds(_NS * rpt, rem)])


def _sc_scatter_sum(table, edge_index, zeros, n):
    d = table.shape[1]
    nacc = n + _SPARE
    e = edge_index.shape[1]
    assert e % _NW == 0
    epw = e // _NW
    assert epw % _C == 0
    nchunk = epw // _C
    tail = nchunk % _NB
    groups = nchunk // _NB
    assert nchunk >= _NB
    rpt, rem = _acc_slabs(nacc)

    mesh = plsc.VectorSubcoreMesh(core_axis_name="c", subcore_axis_name="s")

    @functools.partial(
        pl.kernel,
        mesh=mesh,
        name="sc_scatter_sum",
        out_type=jax.ShapeDtypeStruct((_NC, n, d), jnp.float32),
        scratch_types=[
            pltpu.VMEM((epw,), jnp.int32),
            pltpu.VMEM((epw,), jnp.int32),
            [pltpu.VMEM((_C, d), jnp.float32)] * _NB,
            [pltpu.SemaphoreType.DMA] * _NB,
            [pltpu.SemaphoreType.DMA] * _NB,
            pltpu.VMEM_SHARED((nacc, d), jnp.float32),
        ],
        compiler_params=pltpu.CompilerParams(use_tc_tiling_on_sc=False),
    )
    def k(table_hbm, ei_hbm, zeros_hbm, out_hbm,
          srcs, dsts, rows, gsem, ssem, acc):
        c = lax.axis_index("c")
        s = lax.axis_index("s")
        w = c * _NS + s

        pltpu.sync_copy(ei_hbm.at[0].at[pl.ds(w * epw, epw)], srcs)
        pltpu.sync_copy(ei_hbm.at[1].at[pl.ds(w * epw, epw)], dsts)
        _zero_acc(zeros_hbm, acc, s, rpt, rem)
        plsc.subcore_barrier()

        def fire_gather(i, b):
            pltpu.async_copy(
                table_hbm.at[srcs.at[pl.ds(i * _C, _C)]], rows[b], gsem[b])

        def wait_gather(i, b):
            pltpu.make_async_copy(
                table_hbm.at[srcs.at[pl.ds(i * _C, _C)]], rows[b], gsem[b]
            ).wait()

        def fire_scatter(i, b):
            pltpu.async_copy(
                rows[b], acc.at[dsts.at[pl.ds(i * _C, _C)]], ssem[b],
                add=True)

        def wait_scatter(i, b):
            pltpu.make_async_copy(
                rows[b], acc.at[dsts.at[pl.ds(i * _C, _C)]], ssem[b]
            ).wait()

        for b in range(_NB):
            fire_gather(b, b)

        @pl.loop(0, groups)
        def _(g):
            i0 = g * _NB
            for b in range(_NB):
                i = i0 + b
                wait_gather(i, b)
                fire_scatter(i, b)
            for b in range(_NB):
                i = i0 + b

                @pl.when(i + _NB < nchunk)
                def _():
                    wait_scatter(i, b)
                    fire_gather(i + _NB, b)

        for b in range(tail):
            i = groups * _NB + b
            wait_gather(i, b)
            fire_scatter(i, b)
        for b in range(_NB):
            i = groups * _NB + b if b < tail else (groups - 1) * _NB + b
            wait_scatter(i, b)

        plsc.subcore_barrier()
        _write_acc(acc, out_hbm, c, s, n)

    return k(table, edge_index, zeros)


def _sc_degree(edge_index, zeros, n):
    d = zeros.shape[1]
    nacc = n + _SPARE
    e = edge_index.shape[1]
    assert e % _NW == 0
    epw = e // _NW
    assert epw % _C == 0
    nchunk = epw // _C
    rpt, rem = _acc_slabs(nacc)

    mesh = plsc.VectorSubcoreMesh(core_axis_name="c", subcore_axis_name="s")

    @functools.partial(
        pl.kernel,
        mesh=mesh,
        name="sc_degree",
        out_type=jax.ShapeDtypeStruct((_NC, n, d), jnp.float32),
        scratch_types=[
            pltpu.VMEM((epw,), jnp.int32),
            pltpu.VMEM((_C, d), jnp.float32),
            pltpu.VMEM_SHARED((nacc, d), jnp.float32),
            pltpu.SemaphoreType.DMA,
        ],
        compiler_params=pltpu.CompilerParams(use_tc_tiling_on_sc=False),
    )
    def k(ei_hbm, zeros_hbm, out_hbm, dsts, ones, acc, ssem):
        c = lax.axis_index("c")
        s = lax.axis_index("s")
        w = c * _NS + s

        pltpu.sync_copy(ei_hbm.at[1].at[pl.ds(w * epw, epw)], dsts)

        @pl.loop(0, _C)
        def _(j):
            ones[j] = jnp.full((d,), 1.0, jnp.float32)

        _zero_acc(zeros_hbm, acc, s, rpt, rem)
        plsc.subcore_barrier()

        @pl.loop(0, nchunk)
        def _(i):
            pltpu.async_copy(ones, acc.at[dsts.at[pl.ds(i * _C, _C)]],
                             ssem, add=True)

        @pl.loop(0, nchunk)
        def _(i):
            pltpu.make_async_copy(
                ones, acc.at[dsts.at[pl.ds(i * _C, _C)]], ssem).wait()

        plsc.subcore_barrier()
        _write_acc(acc, out_hbm, c, s, n)

    return k(edge_index, zeros)


_BLK = 1000


def _tc_layer1(x, w1, degp):
    n, din = x.shape
    dh = w1.shape[1]
    dd = degp.shape[2]

    def body(x_ref, w_ref, dp_ref, y_ref, dinv_ref):
        deg = dp_ref[0, :, 0:1] + dp_ref[1, :, 0:1] + 1.0
        dinv = lax.rsqrt(deg)
        dinv_ref[...] = dinv
        xw = jnp.dot(x_ref[...], w_ref[...],
                     preferred_element_type=jnp.float32)
        y_ref[...] = xw * dinv

    grid = n // _BLK
    return pl.pallas_call(
        body,
        name="tc_layer1",
        grid=(grid,),
        in_specs=[
            pl.BlockSpec((_BLK, din), lambda i: (i, 0)),
            pl.BlockSpec((din, dh), lambda i: (0, 0)),
            pl.BlockSpec((_NC, _BLK, dd), lambda i: (0, i, 0)),
        ],
        out_specs=(
            pl.BlockSpec((_BLK, dh), lambda i: (i, 0)),
            pl.BlockSpec((_BLK, 1), lambda i: (i, 0)),
        ),
        out_shape=(
            jax.ShapeDtypeStruct((n, dh), jnp.float32),
            jax.ShapeDtypeStruct((n, 1), jnp.float32),
        ),
    )(x, w1, degp)


def _tc_mid(a, y1, dinv, b1):
    n, dh = y1.shape

    def body(a_ref, y_ref, dinv_ref, b_ref, u_ref):
        total = a_ref[0] + a_ref[1] + y_ref[...]
        h = jnp.maximum(dinv_ref[...] * total + b_ref[...], 0.0)
        u_ref[...] = dinv_ref[...] * h

    grid = n // _BLK
    return pl.pallas_call(
        body,
        name="tc_mid",
        grid=(grid,),
        in_specs=[
            pl.BlockSpec((_NC, _BLK, dh), lambda i: (0, i, 0)),
            pl.BlockSpec((_BLK, dh), lambda i: (i, 0)),
            pl.BlockSpec((_BLK, 1), lambda i: (i, 0)),
            pl.BlockSpec((1, dh), lambda i: (0, 0)),
        ],
        out_specs=pl.BlockSpec((_BLK, dh), lambda i: (i, 0)),
        out_shape=jax.ShapeDtypeStruct((n, dh), jnp.float32),
    )(a, y1, dinv, b1)


def _tc_out(a, u, dinv, w2, b2):
    n, dh = u.shape
    dout = w2.shape[1]

    def body(a_ref, u_ref, dinv_ref, w_ref, b_ref, o_ref):
        m = dinv_ref[...] * (a_ref[0] + a_ref[1] + u_ref[...])
        o_ref[...] = jnp.dot(m, w_ref[...],
                             preferred_element_type=jnp.float32) + b_ref[...]

    grid = n // _BLK
    return pl.pallas_call(
        body,
        name="tc_out",
        grid=(grid,),
        in_specs=[
            pl.BlockSpec((_NC, _BLK, dh), lambda i: (0, i, 0)),
            pl.BlockSpec((_BLK, dh), lambda i: (i, 0)),
            pl.BlockSpec((_BLK, 1), lambda i: (i, 0)),
            pl.BlockSpec((dh, dout), lambda i: (0, 0)),
            pl.BlockSpec((1, dout), lambda i: (0, 0)),
        ],
        out_specs=pl.BlockSpec((_BLK, dout), lambda i: (i, 0)),
        out_shape=jax.ShapeDtypeStruct((n, dout), jnp.float32),
    )(a, u, dinv, w2, b2)


def kernel(x, edge_index, W1, b1, W2, b2):
    n = x.shape[0]
    dh = W1.shape[1]

    ei, _ = _pad_edges(edge_index, n)

    rpt, rem = _acc_slabs(n + _SPARE)
    zrows = max(rpt, rem)
    zeros_h = jnp.zeros((zrows, dh), jnp.float32)
    zeros_d = jnp.zeros((zrows, 16), jnp.float32)

    degp = _sc_degree(ei, zeros_d, n)

    y1, dinv = _tc_layer1(x, W1, degp)
    a1 = _sc_scatter_sum(y1, ei, zeros_h, n)
    u = _tc_mid(a1, y1, dinv, b1.reshape(1, dh))
    a2 = _sc_scatter_sum(u, ei, zeros_h, n)
    return _tc_out(a2, u, dinv, W2, b2.reshape(1, W2.shape[1]))


# speedup vs baseline: 1.8388x; 1.0397x over previous
import functools

import jax
import jax.numpy as jnp
from jax import lax
from jax.experimental import pallas as pl
from jax.experimental.pallas import tpu as pltpu
from jax.experimental.pallas import tpu_sc as plsc

_NC = 2
_NS = 16
_NW = _NC * _NS
_C = 80
_SPARE = 128
_NB = 8


def _pad_edges(edge_index, n):
    e = edge_index.shape[1]
    quantum = _NW * _C
    e_pad = -(-e // quantum) * quantum
    if e_pad == e:
        return edge_index, e
    extra = e_pad - e
    dummy = jnp.stack([
        jnp.zeros((extra,), edge_index.dtype),
        n + (jnp.arange(extra, dtype=edge_index.dtype) % _SPARE),
    ])
    return jnp.concatenate([edge_index, dummy], axis=1), e_pad


def _acc_slabs(nacc):
    rpt = (nacc // _NS) // 8 * 8
    rem = nacc - _NS * rpt
    assert rem % 8 == 0
    return rpt, rem


def _zero_acc(zeros_hbm, acc, s, rpt, rem):
    pltpu.sync_copy(zeros_hbm.at[pl.ds(0, rpt)],
                    acc.at[pl.ds(s * rpt, rpt)])
    if rem:
        @pl.when(s == _NS - 1)
        def _():
            pltpu.sync_copy(zeros_hbm.at[pl.ds(0, rem)],
                            acc.at[pl.ds(_NS * rpt, rem)])


def _write_acc(acc, out_hbm, c, s, n):
    rpt, rem = _acc_slabs(n)
    pltpu.sync_copy(acc.at[pl.ds(s * rpt, rpt)],
                    out_hbm.at[c].at[pl.ds(s * rpt, rpt)])
    if rem:
        @pl.when(s == _NS - 1)
        def _():
            pltpu.sync_copy(acc.at[pl.ds(_NS * rpt, rem)],
                            out_hbm.at[c].at[pl.ds(_NS * rpt, rem)])


def _sc_scatter_sum(table, edge_index, zeros, n):
    d = table.shape[1]
    nacc = n + _SPARE
    e = edge_index.shape[1]
    assert e % _NW == 0
    epw = e // _NW
    assert epw % _C == 0
    nchunk = epw // _C
    tail = nchunk % _NB
    groups = nchunk // _NB
    assert nchunk >= _NB
    rpt, rem = _acc_slabs(nacc)

    mesh = plsc.VectorSubcoreMesh(core_axis_name="c", subcore_axis_name="s")

    @functools.partial(
        pl.kernel,
        mesh=mesh,
        name="sc_scatter_sum",
        out_type=jax.ShapeDtypeStruct((_NC, n, d), jnp.float32),
        scratch_types=[
            pltpu.VMEM((epw,), jnp.int32),
            pltpu.VMEM((epw,), jnp.int32),
            [pltpu.VMEM((_C, d), jnp.float32)] * _NB,
            [pltpu.SemaphoreType.DMA] * _NB,
            [pltpu.SemaphoreType.DMA] * _NB,
            pltpu.VMEM_SHARED((nacc, d), jnp.float32),
        ],
        compiler_params=pltpu.CompilerParams(use_tc_tiling_on_sc=False),
    )
    def k(table_hbm, ei_hbm, zeros_hbm, out_hbm,
          srcs, dsts, rows, gsem, ssem, acc):
        c = lax.axis_index("c")
        s = lax.axis_index("s")
        w = c * _NS + s

        pltpu.sync_copy(ei_hbm.at[0].at[pl.ds(w * epw, epw)], srcs)
        pltpu.sync_copy(ei_hbm.at[1].at[pl.ds(w * epw, epw)], dsts)
        _zero_acc(zeros_hbm, acc, s, rpt, rem)
        plsc.subcore_barrier()

        def fire_gather(i, b):
            pltpu.async_copy(
                table_hbm.at[srcs.at[pl.ds(i * _C, _C)]], rows[b], gsem[b])

        def wait_gather(i, b):
            pltpu.make_async_copy(
                table_hbm.at[srcs.at[pl.ds(i * _C, _C)]], rows[b], gsem[b]
            ).wait()

        def fire_scatter(i, b):
            pltpu.async_copy(
                rows[b], acc.at[dsts.at[pl.ds(i * _C, _C)]], ssem[b],
                add=True)

        def wait_scatter(i, b):
            pltpu.make_async_copy(
                rows[b], acc.at[dsts.at[pl.ds(i * _C, _C)]], ssem[b]
            ).wait()

        for b in range(_NB):
            fire_gather(b, b)

        @pl.loop(0, groups)
        def _(g):
            i0 = g * _NB
            for b in range(_NB):
                i = i0 + b
                wait_gather(i, b)
                fire_scatter(i, b)
            for b in range(_NB):
                i = i0 + b

                @pl.when(i + _NB < nchunk)
                def _():
                    wait_scatter(i, b)
                    fire_gather(i + _NB, b)

        for b in range(tail):
            i = groups * _NB + b
            wait_gather(i, b)
            fire_scatter(i, b)
        for b in range(_NB):
            i = groups * _NB + b if b < tail else (groups - 1) * _NB + b
            wait_scatter(i, b)

        plsc.subcore_barrier()
        _write_acc(acc, out_hbm, c, s, n)

    return k(table, edge_index, zeros)


def _sc_degree(edge_index, zeros, n):
    nacc = n + _SPARE
    e = edge_index.shape[1]
    assert e % _NW == 0
    epw = e // _NW
    assert epw % _C == 0
    nchunk = epw // _C
    rpt, rem = _acc_slabs(nacc)
    opt, orem = _acc_slabs(n)

    mesh = plsc.VectorSubcoreMesh(core_axis_name="c", subcore_axis_name="s")

    @functools.partial(
        pl.kernel,
        mesh=mesh,
        name="sc_degree",
        out_type=jax.ShapeDtypeStruct((_NC, n), jnp.float32),
        scratch_types=[
            pltpu.VMEM((epw,), jnp.int32),
            pltpu.VMEM((_C,), jnp.float32),
            pltpu.VMEM_SHARED((nacc,), jnp.float32),
            pltpu.SemaphoreType.DMA,
        ],
        compiler_params=pltpu.CompilerParams(use_tc_tiling_on_sc=False),
    )
    def k(ei_hbm, zeros_hbm, out_hbm, dsts, ones, acc, ssem):
        c = lax.axis_index("c")
        s = lax.axis_index("s")
        w = c * _NS + s

        pltpu.sync_copy(ei_hbm.at[1].at[pl.ds(w * epw, epw)], dsts)

        @pl.loop(0, _C, step=16)
        def _(j):
            ones[pl.ds(j, 16)] = jnp.full((16,), 1.0, jnp.float32)

        pltpu.sync_copy(zeros_hbm.at[pl.ds(0, rpt)],
                        acc.at[pl.ds(s * rpt, rpt)])
        if rem:
            @pl.when(s == _NS - 1)
            def _():
                pltpu.sync_copy(zeros_hbm.at[pl.ds(0, rem)],
                                acc.at[pl.ds(_NS * rpt, rem)])
        plsc.subcore_barrier()

        @pl.loop(0, nchunk)
        def _(i):
            pltpu.async_copy(ones, acc.at[dsts.at[pl.ds(i * _C, _C)]],
                             ssem, add=True)

        @pl.loop(0, nchunk)
        def _(i):
            pltpu.make_async_copy(
                ones, acc.at[dsts.at[pl.ds(i * _C, _C)]], ssem).wait()

        plsc.subcore_barrier()
        pltpu.sync_copy(acc.at[pl.ds(s * opt, opt)],
                        out_hbm.at[c].at[pl.ds(s * opt, opt)])
        if orem:
            @pl.when(s == _NS - 1)
            def _():
                pltpu.sync_copy(acc.at[pl.ds(_NS * opt, orem)],
                                out_hbm.at[c].at[pl.ds(_NS * opt, orem)])

    return k(edge_index, zeros)


_BLK = 1000


def _tc_dinv(degp):
    n = degp.shape[1]

    def body(dp_ref, dinv_ref):
        deg = dp_ref[0] + dp_ref[1] + 1.0
        dinv_ref[...] = jnp.reshape(lax.rsqrt(deg), (n, 1))

    return pl.pallas_call(
        body,
        name="tc_dinv",
        out_shape=jax.ShapeDtypeStruct((n, 1), jnp.float32),
    )(degp)


def _tc_layer1(x, w1, dinv):
    n, din = x.shape
    dh = w1.shape[1]

    def body(x_ref, w_ref, dinv_ref, y_ref):
        xw = jnp.dot(x_ref[...], w_ref[...],
                     preferred_element_type=jnp.float32)
        y_ref[...] = xw * dinv_ref[...]

    grid = n // _BLK
    return pl.pallas_call(
        body,
        name="tc_layer1",
        grid=(grid,),
        in_specs=[
            pl.BlockSpec((_BLK, din), lambda i: (i, 0)),
            pl.BlockSpec((din, dh), lambda i: (0, 0)),
            pl.BlockSpec((_BLK, 1), lambda i: (i, 0)),
        ],
        out_specs=pl.BlockSpec((_BLK, dh), lambda i: (i, 0)),
        out_shape=jax.ShapeDtypeStruct((n, dh), jnp.float32),
    )(x, w1, dinv)


def _tc_mid(a, y1, dinv, b1):
    n, dh = y1.shape

    def body(a_ref, y_ref, dinv_ref, b_ref, u_ref):
        total = a_ref[0] + a_ref[1] + y_ref[...]
        h = jnp.maximum(dinv_ref[...] * total + b_ref[...], 0.0)
        u_ref[...] = dinv_ref[...] * h

    grid = n // _BLK
    return pl.pallas_call(
        body,
        name="tc_mid",
        grid=(grid,),
        in_specs=[
            pl.BlockSpec((_NC, _BLK, dh), lambda i: (0, i, 0)),
            pl.BlockSpec((_BLK, dh), lambda i: (i, 0)),
            pl.BlockSpec((_BLK, 1), lambda i: (i, 0)),
            pl.BlockSpec((1, dh), lambda i: (0, 0)),
        ],
        out_specs=pl.BlockSpec((_BLK, dh), lambda i: (i, 0)),
        out_shape=jax.ShapeDtypeStruct((n, dh), jnp.float32),
    )(a, y1, dinv, b1)


def _tc_out(a, u, dinv, w2, b2):
    n, dh = u.shape
    dout = w2.shape[1]

    def body(a_ref, u_ref, dinv_ref, w_ref, b_ref, o_ref):
        m = dinv_ref[...] * (a_ref[0] + a_ref[1] + u_ref[...])
        o_ref[...] = jnp.dot(m, w_ref[...],
                             preferred_element_type=jnp.float32) + b_ref[...]

    grid = n // _BLK
    return pl.pallas_call(
        body,
        name="tc_out",
        grid=(grid,),
        in_specs=[
            pl.BlockSpec((_NC, _BLK, dh), lambda i: (0, i, 0)),
            pl.BlockSpec((_BLK, dh), lambda i: (i, 0)),
            pl.BlockSpec((_BLK, 1), lambda i: (i, 0)),
            pl.BlockSpec((dh, dout), lambda i: (0, 0)),
            pl.BlockSpec((1, dout), lambda i: (0, 0)),
        ],
        out_specs=pl.BlockSpec((_BLK, dout), lambda i: (i, 0)),
        out_shape=jax.ShapeDtypeStruct((n, dout), jnp.float32),
    )(a, u, dinv, w2, b2)


def kernel(x, edge_index, W1, b1, W2, b2):
    n = x.shape[0]
    dh = W1.shape[1]

    ei, _ = _pad_edges(edge_index, n)

    rpt, rem = _acc_slabs(n + _SPARE)
    zrows = max(rpt, rem)
    zeros_h = jnp.zeros((zrows, dh), jnp.float32)
    zeros_d = jnp.zeros((zrows,), jnp.float32)

    degp = _sc_degree(ei, zeros_d, n)

    dinv = _tc_dinv(degp)
    y1 = _tc_layer1(x, W1, dinv)
    a1 = _sc_scatter_sum(y1, ei, zeros_h, n)
    u = _tc_mid(a1, y1, dinv, b1.reshape(1, dh))
    a2 = _sc_scatter_sum(u, ei, zeros_h, n)
    return _tc_out(a2, u, dinv, W2, b2.reshape(1, W2.shape[1]))


# speedup vs baseline: 2.1029x; 1.1436x over previous
import functools

import jax
import jax.numpy as jnp
from jax import lax
from jax.experimental import pallas as pl
from jax.experimental.pallas import tpu as pltpu
from jax.experimental.pallas import tpu_sc as plsc

_NC = 2
_NS = 16
_NW = _NC * _NS
_C = 80
_SPARE = 128
_NB = 8


def _pad_edges(edge_index, n):
    e = edge_index.shape[1]
    quantum = _NW * _C
    e_pad = -(-e // quantum) * quantum
    if e_pad == e:
        return edge_index, e
    extra = e_pad - e
    dummy = jnp.stack([
        jnp.zeros((extra,), edge_index.dtype),
        n + (jnp.arange(extra, dtype=edge_index.dtype) % _SPARE),
    ])
    return jnp.concatenate([edge_index, dummy], axis=1), e_pad


def _acc_slabs(nacc):
    rpt = (nacc // _NS) // 8 * 8
    rem = nacc - _NS * rpt
    assert rem % 8 == 0
    return rpt, rem


def _zero_acc(zeros_hbm, acc, s, rpt, rem):
    pltpu.sync_copy(zeros_hbm.at[pl.ds(0, rpt)],
                    acc.at[pl.ds(s * rpt, rpt)])
    if rem:
        @pl.when(s == _NS - 1)
        def _():
            pltpu.sync_copy(zeros_hbm.at[pl.ds(0, rem)],
                            acc.at[pl.ds(_NS * rpt, rem)])


def _write_acc(acc, out_hbm, c, s, n):
    rpt, rem = _acc_slabs(n)
    pltpu.sync_copy(acc.at[pl.ds(s * rpt, rpt)],
                    out_hbm.at[c].at[pl.ds(s * rpt, rpt)])
    if rem:
        @pl.when(s == _NS - 1)
        def _():
            pltpu.sync_copy(acc.at[pl.ds(_NS * rpt, rem)],
                            out_hbm.at[c].at[pl.ds(_NS * rpt, rem)])


def _sc_scatter_sum(table, edge_index, zeros, n):
    d = table.shape[1]
    nacc = n + _SPARE
    e = edge_index.shape[1]
    assert e % _NW == 0
    epw = e // _NW
    assert epw % _C == 0
    nchunk = epw // _C
    tail = nchunk % _NB
    groups = nchunk // _NB
    assert nchunk >= _NB
    rpt, rem = _acc_slabs(nacc)

    mesh = plsc.VectorSubcoreMesh(core_axis_name="c", subcore_axis_name="s")

    @functools.partial(
        pl.kernel,
        mesh=mesh,
        name="sc_scatter_sum",
        out_type=jax.ShapeDtypeStruct((_NC, n, d), jnp.float32),
        scratch_types=[
            pltpu.VMEM((epw,), jnp.int32),
            pltpu.VMEM((epw,), jnp.int32),
            [pltpu.VMEM((_C, d), jnp.float32)] * _NB,
            [pltpu.SemaphoreType.DMA] * _NB,
            [pltpu.SemaphoreType.DMA] * _NB,
            pltpu.VMEM_SHARED((nacc, d), jnp.float32),
        ],
        compiler_params=pltpu.CompilerParams(use_tc_tiling_on_sc=False),
    )
    def k(table_hbm, ei_hbm, zeros_hbm, out_hbm,
          srcs, dsts, rows, gsem, ssem, acc):
        c = lax.axis_index("c")
        s = lax.axis_index("s")
        w = c * _NS + s

        pltpu.sync_copy(ei_hbm.at[0].at[pl.ds(w * epw, epw)], srcs)
        pltpu.sync_copy(ei_hbm.at[1].at[pl.ds(w * epw, epw)], dsts)
        _zero_acc(zeros_hbm, acc, s, rpt, rem)
        plsc.subcore_barrier()

        def fire_gather(i, b):
            pltpu.async_copy(
                table_hbm.at[srcs.at[pl.ds(i * _C, _C)]], rows[b], gsem[b])

        def wait_gather(i, b):
            pltpu.make_async_copy(
                table_hbm.at[srcs.at[pl.ds(i * _C, _C)]], rows[b], gsem[b]
            ).wait()

        def fire_scatter(i, b):
            pltpu.async_copy(
                rows[b], acc.at[dsts.at[pl.ds(i * _C, _C)]], ssem[b],
                add=True)

        def wait_scatter(i, b):
            pltpu.make_async_copy(
                rows[b], acc.at[dsts.at[pl.ds(i * _C, _C)]], ssem[b]
            ).wait()

        for b in range(_NB):
            fire_gather(b, b)

        @pl.loop(0, groups)
        def _(g):
            i0 = g * _NB
            for b in range(_NB):
                i = i0 + b
                wait_gather(i, b)
                fire_scatter(i, b)
            for b in range(_NB):
                i = i0 + b

                @pl.when(i + _NB < nchunk)
                def _():
                    wait_scatter(i, b)
                    fire_gather(i + _NB, b)

        for b in range(tail):
            i = groups * _NB + b
            wait_gather(i, b)
            fire_scatter(i, b)
        for b in range(_NB):
            i = groups * _NB + b if b < tail else (groups - 1) * _NB + b
            wait_scatter(i, b)

        plsc.subcore_barrier()
        _write_acc(acc, out_hbm, c, s, n)

    return k(table, edge_index, zeros)


def _sc_degree(edge_index, zeros, n):
    nacc = n + _SPARE
    e = edge_index.shape[1]
    assert e % _NW == 0
    epw = e // _NW
    assert epw % _C == 0
    nchunk = epw // _C
    rpt, rem = _acc_slabs(nacc)
    opt, orem = _acc_slabs(n)

    mesh = plsc.VectorSubcoreMesh(core_axis_name="c", subcore_axis_name="s")

    @functools.partial(
        pl.kernel,
        mesh=mesh,
        name="sc_degree",
        out_type=jax.ShapeDtypeStruct((_NC, n), jnp.float32),
        scratch_types=[
            pltpu.VMEM((epw,), jnp.int32),
            pltpu.VMEM((_C,), jnp.float32),
            pltpu.VMEM_SHARED((nacc,), jnp.float32),
            pltpu.SemaphoreType.DMA,
        ],
        compiler_params=pltpu.CompilerParams(use_tc_tiling_on_sc=False),
    )
    def k(ei_hbm, zeros_hbm, out_hbm, dsts, ones, acc, ssem):
        c = lax.axis_index("c")
        s = lax.axis_index("s")
        w = c * _NS + s

        pltpu.sync_copy(ei_hbm.at[1].at[pl.ds(w * epw, epw)], dsts)

        @pl.loop(0, _C, step=16)
        def _(j):
            ones[pl.ds(j, 16)] = jnp.full((16,), 1.0, jnp.float32)

        pltpu.sync_copy(zeros_hbm.at[pl.ds(0, rpt)],
                        acc.at[pl.ds(s * rpt, rpt)])
        if rem:
            @pl.when(s == _NS - 1)
            def _():
                pltpu.sync_copy(zeros_hbm.at[pl.ds(0, rem)],
                                acc.at[pl.ds(_NS * rpt, rem)])
        plsc.subcore_barrier()

        @pl.loop(0, nchunk)
        def _(i):
            pltpu.async_copy(ones, acc.at[dsts.at[pl.ds(i * _C, _C)]],
                             ssem, add=True)

        @pl.loop(0, nchunk)
        def _(i):
            pltpu.make_async_copy(
                ones, acc.at[dsts.at[pl.ds(i * _C, _C)]], ssem).wait()

        plsc.subcore_barrier()
        pltpu.sync_copy(acc.at[pl.ds(s * opt, opt)],
                        out_hbm.at[c].at[pl.ds(s * opt, opt)])
        if orem:
            @pl.when(s == _NS - 1)
            def _():
                pltpu.sync_copy(acc.at[pl.ds(_NS * opt, orem)],
                                out_hbm.at[c].at[pl.ds(_NS * opt, orem)])

    return k(edge_index, zeros)


_BLK = 1000


def _tc_dinv(degp):
    n = degp.shape[1]

    def body(dp_ref, dinv_ref):
        deg = dp_ref[0] + dp_ref[1] + 1.0
        dinv_ref[...] = jnp.reshape(lax.rsqrt(deg), (n, 1))

    return pl.pallas_call(
        body,
        name="tc_dinv",
        out_shape=jax.ShapeDtypeStruct((n, 1), jnp.float32),
    )(degp)


def _tc_layer1(x, w1, dinv):
    n, din = x.shape
    dh = w1.shape[1]

    def body(x_ref, w_ref, dinv_ref, y_ref):
        xw = jnp.dot(x_ref[...], w_ref[...],
                     preferred_element_type=jnp.float32)
        y_ref[...] = xw * dinv_ref[...]

    grid = n // _BLK
    return pl.pallas_call(
        body,
        name="tc_layer1",
        grid=(grid,),
        in_specs=[
            pl.BlockSpec((_BLK, din), lambda i: (i, 0)),
            pl.BlockSpec((din, dh), lambda i: (0, 0)),
            pl.BlockSpec((_BLK, 1), lambda i: (i, 0)),
        ],
        out_specs=pl.BlockSpec((_BLK, dh), lambda i: (i, 0)),
        out_shape=jax.ShapeDtypeStruct((n, dh), jnp.float32),
    )(x, w1, dinv)


def _tc_mid(a_p, y1_p, dinv_p, b1_p):
    np2 = y1_p.shape[0]
    blk = 1000

    def body(a_ref, y_ref, dinv_ref, b_ref, u_ref):
        total = a_ref[0] + a_ref[1] + y_ref[...]
        h = jnp.maximum(dinv_ref[...] * total + b_ref[...], 0.0)
        u_ref[...] = dinv_ref[...] * h

    grid = np2 // blk
    return pl.pallas_call(
        body,
        name="tc_mid",
        grid=(grid,),
        in_specs=[
            pl.BlockSpec((_NC, blk, 128), lambda i: (0, i, 0)),
            pl.BlockSpec((blk, 128), lambda i: (i, 0)),
            pl.BlockSpec((blk, 128), lambda i: (i, 0)),
            pl.BlockSpec((1, 128), lambda i: (0, 0)),
        ],
        out_specs=pl.BlockSpec((blk, 128), lambda i: (i, 0)),
        out_shape=jax.ShapeDtypeStruct((np2, 128), jnp.float32),
    )(a_p, y1_p, dinv_p, b1_p)


def _tc_out(a_p, u_p, dinv_p, w2s, b2_p):
    np2 = u_p.shape[0]
    blk = 1000
    dout2 = w2s.shape[1]

    def body(a_ref, u_ref, dinv_ref, w_ref, b_ref, o_ref):
        m = dinv_ref[...] * (a_ref[0] + a_ref[1] + u_ref[...])
        o_ref[...] = jnp.dot(m, w_ref[...],
                             preferred_element_type=jnp.float32) + b_ref[...]

    grid = np2 // blk
    return pl.pallas_call(
        body,
        name="tc_out",
        grid=(grid,),
        in_specs=[
            pl.BlockSpec((_NC, blk, 128), lambda i: (0, i, 0)),
            pl.BlockSpec((blk, 128), lambda i: (i, 0)),
            pl.BlockSpec((blk, 128), lambda i: (i, 0)),
            pl.BlockSpec((128, dout2), lambda i: (0, 0)),
            pl.BlockSpec((1, dout2), lambda i: (0, 0)),
        ],
        out_specs=pl.BlockSpec((blk, dout2), lambda i: (i, 0)),
        out_shape=jax.ShapeDtypeStruct((np2, dout2), jnp.float32),
    )(a_p, u_p, dinv_p, w2s, b2_p)


def kernel(x, edge_index, W1, b1, W2, b2):
    n = x.shape[0]
    dh = W1.shape[1]

    ei, _ = _pad_edges(edge_index, n)

    rpt, rem = _acc_slabs(n + _SPARE)
    zrows = max(rpt, rem)
    zeros_h = jnp.zeros((zrows, dh), jnp.float32)
    zeros_d = jnp.zeros((zrows,), jnp.float32)

    degp = _sc_degree(ei, zeros_d, n)

    dinv = _tc_dinv(degp)
    dinv_p = jnp.broadcast_to(dinv, (n, dh)).reshape(n // 2, 2 * dh)
    y1 = _tc_layer1(x, W1, dinv)
    y1_p = y1.reshape(n // 2, 2 * dh)
    a1 = _sc_scatter_sum(y1_p.reshape(n, dh), ei, zeros_h, n)
    u_p = _tc_mid(a1.reshape(_NC, n // 2, 2 * dh), y1_p, dinv_p,
                  jnp.concatenate([b1, b1]).reshape(1, 2 * dh))
    a2 = _sc_scatter_sum(u_p.reshape(n, dh), ei, zeros_h, n)

    dout = W2.shape[1]
    w2s = jnp.zeros((2 * dh, 2 * dout), jnp.float32)
    w2s = w2s.at[:dh, :dout].set(W2).at[dh:, dout:].set(W2)
    b2_p = jnp.concatenate([b2, b2]).reshape(1, 2 * dout)
    out_p = _tc_out(a2.reshape(_NC, n // 2, 2 * dh), u_p, dinv_p, w2s, b2_p)
    return out_p.reshape(n, dout)


# speedup vs baseline: 2.2187x; 1.0551x over previous
import functools

import jax
import jax.numpy as jnp
from jax import lax
from jax.experimental import pallas as pl
from jax.experimental.pallas import tpu as pltpu
from jax.experimental.pallas import tpu_sc as plsc

_NC = 2
_NS = 16
_NW = _NC * _NS
_C = 80
_SPARE = 128
_NB = 8


def _pad_edges(edge_index, n):
    e = edge_index.shape[1]
    quantum = _NW * _C
    e_pad = -(-e // quantum) * quantum
    if e_pad == e:
        return edge_index, e
    extra = e_pad - e
    dummy = jnp.stack([
        jnp.zeros((extra,), edge_index.dtype),
        n + (jnp.arange(extra, dtype=edge_index.dtype) % _SPARE),
    ])
    return jnp.concatenate([edge_index, dummy], axis=1), e_pad


def _acc_slabs(nacc):
    rpt = (nacc // _NS) // 8 * 8
    rem = nacc - _NS * rpt
    assert rem % 8 == 0
    return rpt, rem


def _zero_acc(zeros_hbm, acc, s, rpt, rem):
    pltpu.sync_copy(zeros_hbm.at[pl.ds(0, rpt)],
                    acc.at[pl.ds(s * rpt, rpt)])
    if rem:
        @pl.when(s == _NS - 1)
        def _():
            pltpu.sync_copy(zeros_hbm.at[pl.ds(0, rem)],
                            acc.at[pl.ds(_NS * rpt, rem)])


def _write_acc(acc, out_hbm, c, s, n):
    rpt, rem = _acc_slabs(n)
    pltpu.sync_copy(acc.at[pl.ds(s * rpt, rpt)],
                    out_hbm.at[c].at[pl.ds(s * rpt, rpt)])
    if rem:
        @pl.when(s == _NS - 1)
        def _():
            pltpu.sync_copy(acc.at[pl.ds(_NS * rpt, rem)],
                            out_hbm.at[c].at[pl.ds(_NS * rpt, rem)])


def _sc_scatter_sum(table, edge_index, zeros, n):
    d = table.shape[1]
    nacc = n + _SPARE
    e = edge_index.shape[1]
    assert e % _NW == 0
    epw = e // _NW
    assert epw % _C == 0
    nchunk = epw // _C
    tail = nchunk % _NB
    groups = nchunk // _NB
    assert nchunk >= _NB
    rpt, rem = _acc_slabs(nacc)

    mesh = plsc.VectorSubcoreMesh(core_axis_name="c", subcore_axis_name="s")

    @functools.partial(
        pl.kernel,
        mesh=mesh,
        name="sc_scatter_sum",
        out_type=jax.ShapeDtypeStruct((_NC, n, d), jnp.float32),
        scratch_types=[
            pltpu.VMEM((epw,), jnp.int32),
            pltpu.VMEM((epw,), jnp.int32),
            [pltpu.VMEM((_C, d), jnp.float32)] * _NB,
            [pltpu.SemaphoreType.DMA] * _NB,
            [pltpu.SemaphoreType.DMA] * _NB,
            pltpu.VMEM_SHARED((nacc, d), jnp.float32),
        ],
        compiler_params=pltpu.CompilerParams(use_tc_tiling_on_sc=False),
    )
    def k(table_hbm, ei_hbm, zeros_hbm, out_hbm,
          srcs, dsts, rows, gsem, ssem, acc):
        c = lax.axis_index("c")
        s = lax.axis_index("s")
        w = c * _NS + s

        pltpu.sync_copy(ei_hbm.at[0].at[pl.ds(w * epw, epw)], srcs)
        pltpu.sync_copy(ei_hbm.at[1].at[pl.ds(w * epw, epw)], dsts)
        _zero_acc(zeros_hbm, acc, s, rpt, rem)
        plsc.subcore_barrier()

        def fire_gather(i, b):
            pltpu.async_copy(
                table_hbm.at[srcs.at[pl.ds(i * _C, _C)]], rows[b], gsem[b])

        def wait_gather(i, b):
            pltpu.make_async_copy(
                table_hbm.at[srcs.at[pl.ds(i * _C, _C)]], rows[b], gsem[b]
            ).wait()

        def fire_scatter(i, b):
            pltpu.async_copy(
                rows[b], acc.at[dsts.at[pl.ds(i * _C, _C)]], ssem[b],
                add=True)

        def wait_scatter(i, b):
            pltpu.make_async_copy(
                rows[b], acc.at[dsts.at[pl.ds(i * _C, _C)]], ssem[b]
            ).wait()

        for b in range(_NB):
            fire_gather(b, b)

        @pl.loop(0, groups - 1)
        def _(g):
            i0 = g * _NB
            for b in range(_NB):
                i = i0 + b
                wait_gather(i, b)
                fire_scatter(i, b)
            for b in range(_NB):
                i = i0 + b
                wait_scatter(i, b)
                fire_gather(i + _NB, b)

        i0 = (groups - 1) * _NB
        for b in range(_NB):
            wait_gather(i0 + b, b)
            fire_scatter(i0 + b, b)
        for b in range(tail):
            wait_scatter(i0 + b, b)
            fire_gather(i0 + _NB + b, b)
        for b in range(tail):
            i = groups * _NB + b
            wait_gather(i, b)
            fire_scatter(i, b)
        for b in range(_NB):
            i = groups * _NB + b if b < tail else (groups - 1) * _NB + b
            wait_scatter(i, b)

        plsc.subcore_barrier()
        _write_acc(acc, out_hbm, c, s, n)

    return k(table, edge_index, zeros)


def _sc_degree(edge_index, zeros, n):
    nacc = n + _SPARE
    e = edge_index.shape[1]
    assert e % _NW == 0
    epw = e // _NW
    assert epw % _C == 0
    nchunk = epw // _C
    rpt, rem = _acc_slabs(nacc)
    opt, orem = _acc_slabs(n)

    mesh = plsc.VectorSubcoreMesh(core_axis_name="c", subcore_axis_name="s")

    @functools.partial(
        pl.kernel,
        mesh=mesh,
        name="sc_degree",
        out_type=jax.ShapeDtypeStruct((_NC, n), jnp.float32),
        scratch_types=[
            pltpu.VMEM((epw,), jnp.int32),
            pltpu.VMEM((_C,), jnp.float32),
            pltpu.VMEM_SHARED((nacc,), jnp.float32),
            pltpu.SemaphoreType.DMA,
        ],
        compiler_params=pltpu.CompilerParams(use_tc_tiling_on_sc=False),
    )
    def k(ei_hbm, zeros_hbm, out_hbm, dsts, ones, acc, ssem):
        c = lax.axis_index("c")
        s = lax.axis_index("s")
        w = c * _NS + s

        pltpu.sync_copy(ei_hbm.at[1].at[pl.ds(w * epw, epw)], dsts)

        @pl.loop(0, _C, step=16)
        def _(j):
            ones[pl.ds(j, 16)] = jnp.full((16,), 1.0, jnp.float32)

        pltpu.sync_copy(zeros_hbm.at[pl.ds(0, rpt)],
                        acc.at[pl.ds(s * rpt, rpt)])
        if rem:
            @pl.when(s == _NS - 1)
            def _():
                pltpu.sync_copy(zeros_hbm.at[pl.ds(0, rem)],
                                acc.at[pl.ds(_NS * rpt, rem)])
        plsc.subcore_barrier()

        @pl.loop(0, nchunk)
        def _(i):
            pltpu.async_copy(ones, acc.at[dsts.at[pl.ds(i * _C, _C)]],
                             ssem, add=True)

        @pl.loop(0, nchunk)
        def _(i):
            pltpu.make_async_copy(
                ones, acc.at[dsts.at[pl.ds(i * _C, _C)]], ssem).wait()

        plsc.subcore_barrier()
        pltpu.sync_copy(acc.at[pl.ds(s * opt, opt)],
                        out_hbm.at[c].at[pl.ds(s * opt, opt)])
        if orem:
            @pl.when(s == _NS - 1)
            def _():
                pltpu.sync_copy(acc.at[pl.ds(_NS * opt, orem)],
                                out_hbm.at[c].at[pl.ds(_NS * opt, orem)])

    return k(edge_index, zeros)


_BLK = 1000


def _tc_layer1(x, w1, degp):
    n, din = x.shape
    dh = w1.shape[1]

    def body(x_ref, w_ref, dp_ref, y_ref, dinv_ref):
        deg = dp_ref[0] + dp_ref[1] + 1.0
        dinv = jnp.reshape(lax.rsqrt(deg), (n, 1))
        dinvb = jnp.broadcast_to(dinv, (n, dh))
        dinv_ref[...] = dinvb
        xw = jnp.dot(x_ref[...], w_ref[...],
                     preferred_element_type=jnp.float32)
        y_ref[...] = xw * dinvb

    return pl.pallas_call(
        body,
        name="tc_layer1",
        out_shape=(
            jax.ShapeDtypeStruct((n, dh), jnp.float32),
            jax.ShapeDtypeStruct((n, dh), jnp.float32),
        ),
    )(x, w1, degp)


def _tc_mid(a_p, y1_p, dinv_p, b1_p):
    np2 = y1_p.shape[0]
    blk = 1000

    def body(a_ref, y_ref, dinv_ref, b_ref, u_ref):
        total = a_ref[0] + a_ref[1] + y_ref[...]
        h = jnp.maximum(dinv_ref[...] * total + b_ref[...], 0.0)
        u_ref[...] = dinv_ref[...] * h

    grid = np2 // blk
    return pl.pallas_call(
        body,
        name="tc_mid",
        grid=(grid,),
        in_specs=[
            pl.BlockSpec((_NC, blk, 128), lambda i: (0, i, 0)),
            pl.BlockSpec((blk, 128), lambda i: (i, 0)),
            pl.BlockSpec((blk, 128), lambda i: (i, 0)),
            pl.BlockSpec((1, 128), lambda i: (0, 0)),
        ],
        out_specs=pl.BlockSpec((blk, 128), lambda i: (i, 0)),
        out_shape=jax.ShapeDtypeStruct((np2, 128), jnp.float32),
    )(a_p, y1_p, dinv_p, b1_p)


def _tc_out(a_p, u_p, dinv_p, w2s, b2_p):
    np2 = u_p.shape[0]
    blk = 1000
    dout2 = w2s.shape[1]

    def body(a_ref, u_ref, dinv_ref, w_ref, b_ref, o_ref):
        m = dinv_ref[...] * (a_ref[0] + a_ref[1] + u_ref[...])
        o_ref[...] = jnp.dot(m, w_ref[...],
                             preferred_element_type=jnp.float32) + b_ref[...]

    grid = np2 // blk
    return pl.pallas_call(
        body,
        name="tc_out",
        grid=(grid,),
        in_specs=[
            pl.BlockSpec((_NC, blk, 128), lambda i: (0, i, 0)),
            pl.BlockSpec((blk, 128), lambda i: (i, 0)),
            pl.BlockSpec((blk, 128), lambda i: (i, 0)),
            pl.BlockSpec((128, dout2), lambda i: (0, 0)),
            pl.BlockSpec((1, dout2), lambda i: (0, 0)),
        ],
        out_specs=pl.BlockSpec((blk, dout2), lambda i: (i, 0)),
        out_shape=jax.ShapeDtypeStruct((np2, dout2), jnp.float32),
    )(a_p, u_p, dinv_p, w2s, b2_p)


def kernel(x, edge_index, W1, b1, W2, b2):
    n = x.shape[0]
    dh = W1.shape[1]

    ei, _ = _pad_edges(edge_index, n)

    rpt, rem = _acc_slabs(n + _SPARE)
    zrows = max(rpt, rem)
    zeros_h = jnp.zeros((zrows, dh), jnp.float32)
    zeros_d = jnp.zeros((zrows,), jnp.float32)

    degp = _sc_degree(ei, zeros_d, n)

    y1, dinvb = _tc_layer1(x, W1, degp)
    dinv_p = dinvb.reshape(n // 2, 2 * dh)
    y1_p = y1.reshape(n // 2, 2 * dh)
    a1 = _sc_scatter_sum(y1_p.reshape(n, dh), ei, zeros_h, n)
    u_p = _tc_mid(a1.reshape(_NC, n // 2, 2 * dh), y1_p, dinv_p,
                  jnp.concatenate([b1, b1]).reshape(1, 2 * dh))
    a2 = _sc_scatter_sum(u_p.reshape(n, dh), ei, zeros_h, n)

    dout = W2.shape[1]
    w2s = jnp.zeros((2 * dh, 2 * dout), jnp.float32)
    w2s = w2s.at[:dh, :dout].set(W2).at[dh:, dout:].set(W2)
    b2_p = jnp.concatenate([b2, b2]).reshape(1, 2 * dout)
    out_p = _tc_out(a2.reshape(_NC, n // 2, 2 * dh), u_p, dinv_p, w2s, b2_p)
    return out_p.reshape(n, dout)
